# Initial kernel scaffold; baseline (speedup 1.0000x reference)
import jax
import jax.numpy as jnp
from jax.experimental import pallas as pl


def kernel(x, sc_conv_w, sc_conv_b, sc_lin_w, sc_lin_b, enc1_w, enc1_b, bn1_g, bn1_b, enc2_w, enc2_b, bn2_g, bn2_b, enc3_w, enc3_b, bn3_g, bn3_b, enc4_w, enc4_b, bn4_g, bn4_b, proj_w, proj_b, cls_w, cls_b):
    raise NotImplementedError("write your pallas kernel here")


# speedup vs baseline: 1.1805x; 1.1805x over previous
import functools

import jax
import jax.numpy as jnp
from jax.experimental import pallas as pl
from jax.experimental.pallas import tpu as pltpu

E = 8
NUM_CLASSES = 10
IN_CH = 3
B = 256
HW = 32
T = 0.1
CAP = 32
CHUNK = 8
N_CHUNKS = B // CHUNK
STEPS_PER_EXPERT = CAP // CHUNK
ROUTE_ROUNDS = 32


def _scores_body(x_ref, w_ref, b_ref, lmat_ref, lb_ref, out_ref):
    xb = x_ref[...]
    n = xb.shape[0]
    xb = xb.reshape(n, IN_CH, HW * HW)
    h = jax.lax.dot_general(xb, w_ref[...], (((1,), (1,)), ((), ())),
                            preferred_element_type=jnp.float32)
    h = jnp.maximum(h + b_ref[...].reshape(1, 1, E * 8), 0.0)
    m = jnp.mean(h, axis=1)
    out_ref[...] = (
        jnp.dot(m, lmat_ref[...], preferred_element_type=jnp.float32)
        + lb_ref[...]
    )


def _route_body(s_ref, g_ref, slot_ref, perm_ref):
    st0 = (s_ref[...] + g_ref[...]) / T
    ltri = (jax.lax.broadcasted_iota(jnp.int32, (B, B), 0)
            >= jax.lax.broadcasted_iota(jnp.int32, (B, B), 1)).astype(jnp.float32)
    e_iota = jax.lax.broadcasted_iota(jnp.int32, (1, E), 1)

    def body(it, state):
        assignment, cap, active = state
        deficit = jnp.clip(cap.astype(jnp.float32) / CAP, 0.0, 1.0)
        stt = st0 * (1.0 - deficit)
        full = cap >= CAP
        stt = jnp.where(full, -jnp.inf, stt)
        mx = jnp.max(stt, axis=1, keepdims=True)
        ismax = stt == mx
        best = jnp.min(jnp.where(ismax, e_iota, E), axis=1, keepdims=True)
        onehot = (e_iota == best).astype(jnp.float32) * active
        order = jnp.dot(ltri, onehot, preferred_element_type=jnp.float32)
        quota = jnp.minimum(jnp.int32(1) << jnp.minimum(it, 30), jnp.int32(CAP))
        space = jnp.minimum(jnp.int32(CAP) - cap, quota).astype(jnp.float32)
        sel = onehot * (order <= space).astype(jnp.float32)
        assignment = jnp.maximum(assignment, sel)
        cap = cap + jnp.sum(sel, axis=0, keepdims=True).astype(jnp.int32)
        active = active * (1.0 - jnp.max(sel, axis=1, keepdims=True))
        return assignment, cap, active

    init = (jnp.zeros((B, E), jnp.float32),
            jnp.zeros((1, E), jnp.int32),
            jnp.ones((B, 1), jnp.float32))
    assignment, _, _ = jax.lax.fori_loop(0, ROUTE_ROUNDS, body, init)

    order = jnp.dot(ltri, assignment, preferred_element_type=jnp.float32)
    eid = jnp.sum(assignment * e_iota.astype(jnp.float32), axis=1, keepdims=True)
    rank = jnp.sum(assignment * order, axis=1, keepdims=True) - 1.0
    slot = (eid * CAP + rank).astype(jnp.int32)
    slot_ref[...] = slot
    row_i = jax.lax.broadcasted_iota(jnp.int32, (B, B), 0).astype(jnp.float32)
    col_p = jax.lax.broadcasted_iota(jnp.int32, (B, B), 1)
    onehot_slot = (slot == col_p).astype(jnp.float32)
    perm_ref[...] = jnp.sum(onehot_slot * row_i, axis=0, keepdims=True).astype(jnp.int32)


def _gather_body(perm_ref, x_ref, out_ref):
    del perm_ref
    out_ref[...] = x_ref[...]


def _scatter_body(slot_ref, lg_ref, out_ref):
    cols = jax.lax.broadcasted_iota(jnp.int32, (B, B), 1)
    onehot = (slot_ref[...] == cols).astype(jnp.float32)
    out_ref[...] = jnp.dot(onehot, lg_ref[...], preferred_element_type=jnp.float32)


def _conv_nhwc(pad_ref, h, w_ref, b_ref, hh, cout):
    n, _, _, cin = h.shape
    pad_ref[...] = jnp.zeros_like(pad_ref)
    pad_ref[:, 1:hh + 1, 1:hh + 1, :] = h
    w = w_ref[0]
    acc = None
    for t in range(9):
        dy, dx = t // 3, t % 3
        sl = pad_ref[:, dy:dy + hh, dx:dx + hh, :].reshape(n * hh * hh, cin)
        r = jnp.dot(sl, w[t], preferred_element_type=jnp.float32)
        acc = r if acc is None else acc + r
    acc = acc.reshape(n, hh, hh, cout) + b_ref[0].reshape(1, 1, 1, cout)
    return jnp.maximum(acc, 0.0)


def _pool2(h):
    n, hh, ww, c = h.shape
    r = h.reshape(n, hh // 2, 2, ww // 2, 2, c)
    return jnp.max(jnp.max(r, axis=4), axis=2)


def _expert_body(xg_ref, w1_ref, b1_ref, w2_ref, b2_ref, w3_ref, b3_ref,
                 w4_ref, b4_ref, pw_ref, pb_ref, cw_ref, cb_ref, out_ref,
                 pad1, pad2, pad3, pad4):
    n = CHUNK
    pad1[...] = jnp.zeros_like(pad1)
    pad1[:, :, 1:HW + 1, 1:HW + 1] = xg_ref[...]
    w1 = w1_ref[0]
    acc = None
    for t in range(9):
        dy, dx = t // 3, t % 3
        sl = pad1[:, :, dy:dy + HW, dx:dx + HW]
        r = jax.lax.dot_general(sl, w1[t], (((1,), (0,)), ((), ())),
                                preferred_element_type=jnp.float32)
        acc = r if acc is None else acc + r
    h = jnp.maximum(acc + b1_ref[0].reshape(1, 1, 1, 64), 0.0)

    h = _conv_nhwc(pad2, h, w2_ref, b2_ref, 32, 64)
    h = _pool2(h)
    h = _conv_nhwc(pad3, h, w3_ref, b3_ref, 16, 128)
    h = _pool2(h)
    h = _conv_nhwc(pad4, h, w4_ref, b4_ref, 8, 256)
    z = jnp.mean(h.reshape(n, 64, 256), axis=1)
    z = jnp.dot(z, pw_ref[0], preferred_element_type=jnp.float32) + pb_ref[0]
    out_ref[...] = (
        jnp.dot(z, cw_ref[0], preferred_element_type=jnp.float32) + cb_ref[0]
    )


def _prep_conv(w, b, g, bb):
    scale = g / jnp.sqrt(1.0 + 1e-5)
    w = w * scale[:, :, None, None, None]
    b = b * scale + bb
    e, cout, cin, _, _ = w.shape
    wt = jnp.transpose(w, (0, 3, 4, 2, 1)).reshape(e, 9, cin, cout)
    return wt, b.reshape(e, 1, cout)


def kernel(x, sc_conv_w, sc_conv_b, sc_lin_w, sc_lin_b,
           enc1_w, enc1_b, bn1_g, bn1_b, enc2_w, enc2_b, bn2_g, bn2_b,
           enc3_w, enc3_b, bn3_g, bn3_b, enc4_w, enc4_b, bn4_g, bn4_b,
           proj_w, proj_b, cls_w, cls_b):
    f32 = jnp.float32

    w_sc = sc_conv_w.reshape(E * 8, IN_CH)
    b_sc = sc_conv_b.reshape(1, E * 8)
    lin = sc_lin_w.reshape(E, 8)
    lmat = (jnp.eye(E, dtype=f32)[:, None, :] * lin[:, :, None]).reshape(E * 8, E)
    lb = sc_lin_b.reshape(1, E)

    scores = pl.pallas_call(
        _scores_body,
        grid=(B // 32,),
        in_specs=[
            pl.BlockSpec((32, IN_CH, HW, HW), lambda i: (i, 0, 0, 0)),
            pl.BlockSpec((E * 8, IN_CH), lambda i: (0, 0)),
            pl.BlockSpec((1, E * 8), lambda i: (0, 0)),
            pl.BlockSpec((E * 8, E), lambda i: (0, 0)),
            pl.BlockSpec((1, E), lambda i: (0, 0)),
        ],
        out_specs=pl.BlockSpec((32, E), lambda i: (i, 0)),
        out_shape=jax.ShapeDtypeStruct((B, E), f32),
    )(x, w_sc, b_sc, lmat, lb)

    gk = jax.random.fold_in(jax.random.key(0), 123)
    u = jax.random.uniform(gk, (B, E), dtype=f32, minval=1e-6, maxval=1.0 - 1e-6)
    gumbel = -jnp.log(-jnp.log(u))

    slot, perm = pl.pallas_call(
        _route_body,
        out_shape=(jax.ShapeDtypeStruct((B, 1), jnp.int32),
                   jax.ShapeDtypeStruct((1, B), jnp.int32)),
    )(scores, gumbel)
    perm_flat = perm.reshape(B)

    xg = pl.pallas_call(
        _gather_body,
        grid_spec=pltpu.PrefetchScalarGridSpec(
            num_scalar_prefetch=1,
            grid=(B,),
            in_specs=[pl.BlockSpec((1, IN_CH, HW, HW),
                                   lambda i, perm_ref: (perm_ref[i], 0, 0, 0))],
            out_specs=pl.BlockSpec((1, IN_CH, HW, HW),
                                   lambda i, perm_ref: (i, 0, 0, 0)),
        ),
        out_shape=jax.ShapeDtypeStruct((B, IN_CH, HW, HW), f32),
    )(perm_flat, x)

    w1p, b1p = _prep_conv(enc1_w, enc1_b, bn1_g, bn1_b)
    w2p, b2p = _prep_conv(enc2_w, enc2_b, bn2_g, bn2_b)
    w3p, b3p = _prep_conv(enc3_w, enc3_b, bn3_g, bn3_b)
    w4p, b4p = _prep_conv(enc4_w, enc4_b, bn4_g, bn4_b)
    pwt = jnp.transpose(proj_w, (0, 2, 1))
    pbp = proj_b.reshape(E, 1, 256)
    cwt = jnp.transpose(cls_w, (0, 2, 1))
    cbp = cls_b.reshape(E, 1, NUM_CLASSES)

    sp = STEPS_PER_EXPERT

    def _wspec(shape3):
        return pl.BlockSpec((1,) + shape3, lambda c: (c // sp, 0, 0, 0))

    def _bspec(cout):
        return pl.BlockSpec((1, 1, cout), lambda c: (c // sp, 0, 0))

    logits_g = pl.pallas_call(
        _expert_body,
        grid=(N_CHUNKS,),
        in_specs=[
            pl.BlockSpec((CHUNK, IN_CH, HW, HW), lambda c: (c, 0, 0, 0)),
            _wspec((9, IN_CH, 64)), _bspec(64),
            _wspec((9, 64, 64)), _bspec(64),
            _wspec((9, 64, 128)), _bspec(128),
            _wspec((9, 128, 256)), _bspec(256),
            pl.BlockSpec((1, 256, 256), lambda c: (c // sp, 0, 0)),
            _bspec(256),
            pl.BlockSpec((1, 256, NUM_CLASSES), lambda c: (c // sp, 0, 0)),
            _bspec(NUM_CLASSES),
        ],
        out_specs=pl.BlockSpec((CHUNK, NUM_CLASSES), lambda c: (c, 0)),
        out_shape=jax.ShapeDtypeStruct((B, NUM_CLASSES), f32),
        scratch_shapes=[
            pltpu.VMEM((CHUNK, IN_CH, HW + 2, HW + 2), f32),
            pltpu.VMEM((CHUNK, HW + 2, HW + 2, 64), f32),
            pltpu.VMEM((CHUNK, 18, 18, 64), f32),
            pltpu.VMEM((CHUNK, 10, 10, 128), f32),
        ],
    )(xg, w1p, b1p, w2p, b2p, w3p, b3p, w4p, b4p, pwt, pbp, cwt, cbp)

    out = pl.pallas_call(
        _scatter_body,
        out_shape=jax.ShapeDtypeStruct((B, NUM_CLASSES), f32),
    )(slot, logits_g)
    return out


# speedup vs baseline: 1.4607x; 1.2374x over previous
import functools

import jax
import jax.numpy as jnp
from jax.experimental import pallas as pl
from jax.experimental.pallas import tpu as pltpu

E = 8
NUM_CLASSES = 10
IN_CH = 3
B = 256
HW = 32
T = 0.1
CAP = 32
CHUNK = 8
N_CHUNKS = B // CHUNK
STEPS_PER_EXPERT = CAP // CHUNK
ROUTE_ROUNDS = 32


def _scores_body(x_ref, w_ref, b_ref, lmat_ref, lb_ref, out_ref):
    xb = x_ref[...]
    n = xb.shape[0]
    xb = xb.reshape(n, IN_CH, HW * HW)
    h = jax.lax.dot_general(xb, w_ref[...], (((1,), (1,)), ((), ())),
                            preferred_element_type=jnp.float32)
    h = jnp.maximum(h + b_ref[...].reshape(1, 1, E * 8), 0.0)
    m = jnp.mean(h, axis=1)
    out_ref[...] = (
        jnp.dot(m, lmat_ref[...], preferred_element_type=jnp.float32)
        + lb_ref[...]
    )


def _route_body(s_ref, g_ref, slot_ref, perm_ref):
    st0 = (s_ref[...] + g_ref[...]) / T
    ltri = (jax.lax.broadcasted_iota(jnp.int32, (B, B), 0)
            >= jax.lax.broadcasted_iota(jnp.int32, (B, B), 1)).astype(jnp.float32)
    e_iota = jax.lax.broadcasted_iota(jnp.int32, (1, E), 1)

    def body(it, state):
        assignment, cap, active = state
        deficit = jnp.clip(cap.astype(jnp.float32) / CAP, 0.0, 1.0)
        stt = st0 * (1.0 - deficit)
        full = cap >= CAP
        stt = jnp.where(full, -jnp.inf, stt)
        mx = jnp.max(stt, axis=1, keepdims=True)
        ismax = stt == mx
        best = jnp.min(jnp.where(ismax, e_iota, E), axis=1, keepdims=True)
        onehot = (e_iota == best).astype(jnp.float32) * active
        order = jnp.dot(ltri, onehot, preferred_element_type=jnp.float32)
        quota = jnp.minimum(jnp.int32(1) << jnp.minimum(it, 30), jnp.int32(CAP))
        space = jnp.minimum(jnp.int32(CAP) - cap, quota).astype(jnp.float32)
        sel = onehot * (order <= space).astype(jnp.float32)
        assignment = jnp.maximum(assignment, sel)
        cap = cap + jnp.sum(sel, axis=0, keepdims=True).astype(jnp.int32)
        active = active * (1.0 - jnp.max(sel, axis=1, keepdims=True))
        return assignment, cap, active

    init = (jnp.zeros((B, E), jnp.float32),
            jnp.zeros((1, E), jnp.int32),
            jnp.ones((B, 1), jnp.float32))
    assignment, _, _ = jax.lax.fori_loop(0, ROUTE_ROUNDS, body, init)

    order = jnp.dot(ltri, assignment, preferred_element_type=jnp.float32)
    eid = jnp.sum(assignment * e_iota.astype(jnp.float32), axis=1, keepdims=True)
    rank = jnp.sum(assignment * order, axis=1, keepdims=True) - 1.0
    slot = (eid * CAP + rank).astype(jnp.int32)
    slot_ref[...] = slot
    row_i = jax.lax.broadcasted_iota(jnp.int32, (B, B), 0).astype(jnp.float32)
    col_p = jax.lax.broadcasted_iota(jnp.int32, (B, B), 1)
    onehot_slot = (slot == col_p).astype(jnp.float32)
    perm_ref[...] = jnp.sum(onehot_slot * row_i, axis=0, keepdims=True).astype(jnp.int32)


def _gather_body(perm_ref, x_ref, out_ref):
    del perm_ref
    t = jax.lax.dot_general(
        x_ref[0].reshape(IN_CH, HW * HW), jnp.eye(IN_CH, dtype=jnp.float32),
        (((0,), (0,)), ((), ())), preferred_element_type=jnp.float32)
    out_ref[...] = jnp.zeros_like(out_ref)
    out_ref[0, 1:HW + 1, 1:HW + 1, :] = t.reshape(HW, HW, IN_CH)


def _scatter_body(slot_ref, lg_ref, out_ref):
    cols = jax.lax.broadcasted_iota(jnp.int32, (B, B), 1)
    onehot = (slot_ref[...] == cols).astype(jnp.float32)
    out_ref[...] = jnp.dot(onehot, lg_ref[...], preferred_element_type=jnp.float32)


def _conv_nhwc(pad_ref, h, w_ref, b_ref, hh, cout):
    n, _, _, cin = h.shape
    pad_ref[...] = jnp.zeros_like(pad_ref)
    pad_ref[:, 1:hh + 1, 1:hh + 1, :] = h
    w = w_ref[0]
    acc = None
    for dx in range(3):
        sx = pad_ref[:, :, dx:dx + hh, :]
        for dy in range(3):
            sl = sx[:, dy:dy + hh, :, :].reshape(n * hh * hh, cin)
            r = jnp.dot(sl, w[3 * dy + dx], preferred_element_type=jnp.float32)
            acc = r if acc is None else acc + r
    acc = acc.reshape(n, hh, hh, cout) + b_ref[0].reshape(1, 1, 1, cout)
    return jnp.maximum(acc, 0.0)


def _pool2(h):
    n, hh, ww, c = h.shape
    r = h.reshape(n, hh // 2, 2, ww // 2, 2, c)
    return jnp.max(jnp.max(r, axis=4), axis=2)


def _expert_body(xg_ref, w1_ref, b1_ref, w2_ref, b2_ref, w3_ref, b3_ref,
                 w4_ref, b4_ref, pw_ref, pb_ref, cw_ref, cb_ref, out_ref,
                 pad2, pad3, pad4):
    n = CHUNK
    w1 = w1_ref[0]
    acc = None
    for dx in range(3):
        sx = xg_ref[:, :, dx:dx + HW, :]
        for dy in range(3):
            sl = sx[:, dy:dy + HW, :, :].reshape(n * HW * HW, IN_CH)
            r = jnp.dot(sl, w1[3 * dy + dx], preferred_element_type=jnp.float32)
            acc = r if acc is None else acc + r
    h = acc.reshape(n, HW, HW, 64) + b1_ref[0].reshape(1, 1, 1, 64)
    h = jnp.maximum(h, 0.0)

    h = _conv_nhwc(pad2, h, w2_ref, b2_ref, 32, 64)
    h = _pool2(h)
    h = _conv_nhwc(pad3, h, w3_ref, b3_ref, 16, 128)
    h = _pool2(h)
    h = _conv_nhwc(pad4, h, w4_ref, b4_ref, 8, 256)
    z = jnp.mean(h.reshape(n, 64, 256), axis=1)
    z = jnp.dot(z, pw_ref[0], preferred_element_type=jnp.float32) + pb_ref[0]
    out_ref[...] = (
        jnp.dot(z, cw_ref[0], preferred_element_type=jnp.float32) + cb_ref[0]
    )


def _prep_conv(w, b, g, bb):
    scale = g / jnp.sqrt(1.0 + 1e-5)
    w = w * scale[:, :, None, None, None]
    b = b * scale + bb
    e, cout, cin, _, _ = w.shape
    wt = jnp.transpose(w, (0, 3, 4, 2, 1)).reshape(e, 9, cin, cout)
    return wt, b.reshape(e, 1, cout)


def kernel(x, sc_conv_w, sc_conv_b, sc_lin_w, sc_lin_b,
           enc1_w, enc1_b, bn1_g, bn1_b, enc2_w, enc2_b, bn2_g, bn2_b,
           enc3_w, enc3_b, bn3_g, bn3_b, enc4_w, enc4_b, bn4_g, bn4_b,
           proj_w, proj_b, cls_w, cls_b):
    f32 = jnp.float32

    w_sc = sc_conv_w.reshape(E * 8, IN_CH)
    b_sc = sc_conv_b.reshape(1, E * 8)
    lin = sc_lin_w.reshape(E, 8)
    lmat = (jnp.eye(E, dtype=f32)[:, None, :] * lin[:, :, None]).reshape(E * 8, E)
    lb = sc_lin_b.reshape(1, E)

    scores = pl.pallas_call(
        _scores_body,
        grid=(B // 32,),
        in_specs=[
            pl.BlockSpec((32, IN_CH, HW, HW), lambda i: (i, 0, 0, 0)),
            pl.BlockSpec((E * 8, IN_CH), lambda i: (0, 0)),
            pl.BlockSpec((1, E * 8), lambda i: (0, 0)),
            pl.BlockSpec((E * 8, E), lambda i: (0, 0)),
            pl.BlockSpec((1, E), lambda i: (0, 0)),
        ],
        out_specs=pl.BlockSpec((32, E), lambda i: (i, 0)),
        out_shape=jax.ShapeDtypeStruct((B, E), f32),
    )(x, w_sc, b_sc, lmat, lb)

    gk = jax.random.fold_in(jax.random.key(0), 123)
    u = jax.random.uniform(gk, (B, E), dtype=f32, minval=1e-6, maxval=1.0 - 1e-6)
    gumbel = -jnp.log(-jnp.log(u))

    slot, perm = pl.pallas_call(
        _route_body,
        out_shape=(jax.ShapeDtypeStruct((B, 1), jnp.int32),
                   jax.ShapeDtypeStruct((1, B), jnp.int32)),
    )(scores, gumbel)
    perm_flat = perm.reshape(B)

    xg = pl.pallas_call(
        _gather_body,
        grid_spec=pltpu.PrefetchScalarGridSpec(
            num_scalar_prefetch=1,
            grid=(B,),
            in_specs=[pl.BlockSpec((1, IN_CH, HW, HW),
                                   lambda i, perm_ref: (perm_ref[i], 0, 0, 0))],
            out_specs=pl.BlockSpec((1, HW + 2, HW + 2, IN_CH),
                                   lambda i, perm_ref: (i, 0, 0, 0)),
        ),
        out_shape=jax.ShapeDtypeStruct((B, HW + 2, HW + 2, IN_CH), f32),
    )(perm_flat, x)

    w1p, b1p = _prep_conv(enc1_w, enc1_b, bn1_g, bn1_b)
    w2p, b2p = _prep_conv(enc2_w, enc2_b, bn2_g, bn2_b)
    w3p, b3p = _prep_conv(enc3_w, enc3_b, bn3_g, bn3_b)
    w4p, b4p = _prep_conv(enc4_w, enc4_b, bn4_g, bn4_b)
    pwt = jnp.transpose(proj_w, (0, 2, 1))
    pbp = proj_b.reshape(E, 1, 256)
    cwt = jnp.transpose(cls_w, (0, 2, 1))
    cbp = cls_b.reshape(E, 1, NUM_CLASSES)

    sp = STEPS_PER_EXPERT

    def _wspec(shape3):
        return pl.BlockSpec((1,) + shape3, lambda c: (c // sp, 0, 0, 0))

    def _bspec(cout):
        return pl.BlockSpec((1, 1, cout), lambda c: (c // sp, 0, 0))

    logits_g = pl.pallas_call(
        _expert_body,
        grid=(N_CHUNKS,),
        in_specs=[
            pl.BlockSpec((CHUNK, HW + 2, HW + 2, IN_CH), lambda c: (c, 0, 0, 0)),
            _wspec((9, IN_CH, 64)), _bspec(64),
            _wspec((9, 64, 64)), _bspec(64),
            _wspec((9, 64, 128)), _bspec(128),
            _wspec((9, 128, 256)), _bspec(256),
            pl.BlockSpec((1, 256, 256), lambda c: (c // sp, 0, 0)),
            _bspec(256),
            pl.BlockSpec((1, 256, NUM_CLASSES), lambda c: (c // sp, 0, 0)),
            _bspec(NUM_CLASSES),
        ],
        out_specs=pl.BlockSpec((CHUNK, NUM_CLASSES), lambda c: (c, 0)),
        out_shape=jax.ShapeDtypeStruct((B, NUM_CLASSES), f32),
        scratch_shapes=[
            pltpu.VMEM((CHUNK, HW + 2, HW + 2, 64), f32),
            pltpu.VMEM((CHUNK, 18, 18, 64), f32),
            pltpu.VMEM((CHUNK, 10, 10, 128), f32),
        ],
    )(xg, w1p, b1p, w2p, b2p, w3p, b3p, w4p, b4p, pwt, pbp, cwt, cbp)

    out = pl.pallas_call(
        _scatter_body,
        out_shape=jax.ShapeDtypeStruct((B, NUM_CLASSES), f32),
    )(slot, logits_g)
    return out


# speedup vs baseline: 2.3535x; 1.6112x over previous
import functools

import jax
import jax.numpy as jnp
from jax.experimental import pallas as pl
from jax.experimental.pallas import tpu as pltpu

E = 8
NUM_CLASSES = 10
IN_CH = 3
B = 256
HW = 32
T = 0.1
CAP = 32
CHUNK = 8
N_CHUNKS = B // CHUNK
STEPS_PER_EXPERT = CAP // CHUNK
ROUTE_ROUNDS = 32


def _scores_body(x_ref, w_ref, b_ref, lmat_ref, lb_ref, out_ref):
    xb = x_ref[...]
    n = xb.shape[0]
    xb = xb.reshape(n, IN_CH, HW * HW)
    h = jax.lax.dot_general(xb, w_ref[...], (((1,), (1,)), ((), ())),
                            preferred_element_type=jnp.float32)
    h = jnp.maximum(h + b_ref[...].reshape(1, 1, E * 8), 0.0)
    m = jnp.mean(h, axis=1)
    out_ref[...] = (
        jnp.dot(m, lmat_ref[...], preferred_element_type=jnp.float32)
        + lb_ref[...]
    )


def _route_body(s_ref, g_ref, slot_ref, perm_ref):
    st0 = (s_ref[...] + g_ref[...]) / T
    ltri = (jax.lax.broadcasted_iota(jnp.int32, (B, B), 0)
            >= jax.lax.broadcasted_iota(jnp.int32, (B, B), 1)).astype(jnp.float32)
    e_iota = jax.lax.broadcasted_iota(jnp.int32, (1, E), 1)

    def body(it, state):
        assignment, cap, active = state
        deficit = jnp.clip(cap.astype(jnp.float32) / CAP, 0.0, 1.0)
        stt = st0 * (1.0 - deficit)
        full = cap >= CAP
        stt = jnp.where(full, -jnp.inf, stt)
        mx = jnp.max(stt, axis=1, keepdims=True)
        ismax = stt == mx
        best = jnp.min(jnp.where(ismax, e_iota, E), axis=1, keepdims=True)
        onehot = (e_iota == best).astype(jnp.float32) * active
        order = jnp.dot(ltri, onehot, preferred_element_type=jnp.float32)
        quota = jnp.minimum(jnp.int32(1) << jnp.minimum(it, 30), jnp.int32(CAP))
        space = jnp.minimum(jnp.int32(CAP) - cap, quota).astype(jnp.float32)
        sel = onehot * (order <= space).astype(jnp.float32)
        assignment = jnp.maximum(assignment, sel)
        cap = cap + jnp.sum(sel, axis=0, keepdims=True).astype(jnp.int32)
        active = active * (1.0 - jnp.max(sel, axis=1, keepdims=True))
        return assignment, cap, active

    init = (jnp.zeros((B, E), jnp.float32),
            jnp.zeros((1, E), jnp.int32),
            jnp.ones((B, 1), jnp.float32))
    assignment, _, _ = jax.lax.fori_loop(0, ROUTE_ROUNDS, body, init)

    order = jnp.dot(ltri, assignment, preferred_element_type=jnp.float32)
    eid = jnp.sum(assignment * e_iota.astype(jnp.float32), axis=1, keepdims=True)
    rank = jnp.sum(assignment * order, axis=1, keepdims=True) - 1.0
    slot = (eid * CAP + rank).astype(jnp.int32)
    slot_ref[...] = slot
    row_i = jax.lax.broadcasted_iota(jnp.int32, (B, B), 0).astype(jnp.float32)
    col_p = jax.lax.broadcasted_iota(jnp.int32, (B, B), 1)
    onehot_slot = (slot == col_p).astype(jnp.float32)
    perm_ref[...] = jnp.sum(onehot_slot * row_i, axis=0, keepdims=True).astype(jnp.int32)


def _scatter_body(slot_ref, lg_ref, out_ref):
    cols = jax.lax.broadcasted_iota(jnp.int32, (B, B), 1)
    onehot = (slot_ref[...] == cols).astype(jnp.float32)
    out_ref[...] = jnp.dot(onehot, lg_ref[...], preferred_element_type=jnp.float32)


def _conv_nhwc(pad_ref, h, w_ref, b_ref, hh, cout):
    n, _, _, cin = h.shape

    @pl.when(pl.program_id(0) == 0)
    def _zero_borders():
        pad_ref[:, 0:1, :, :] = jnp.zeros((n, 1, hh + 2, cin), jnp.float32)
        pad_ref[:, hh + 1:hh + 2, :, :] = jnp.zeros((n, 1, hh + 2, cin), jnp.float32)
        pad_ref[:, 1:hh + 1, 0:1, :] = jnp.zeros((n, hh, 1, cin), jnp.float32)
        pad_ref[:, 1:hh + 1, hh + 1:hh + 2, :] = jnp.zeros((n, hh, 1, cin), jnp.float32)

    pad_ref[:, 1:hh + 1, 1:hh + 1, :] = h
    w = w_ref[0]
    acc = None
    for dx in range(3):
        sx = pad_ref[:, :, dx:dx + hh, :]
        for dy in range(3):
            sl = sx[:, dy:dy + hh, :, :].reshape(n * hh * hh, cin)
            r = jnp.dot(sl, w[3 * dy + dx], preferred_element_type=jnp.float32)
            acc = r if acc is None else acc + r
    acc = acc.reshape(n, hh, hh, cout) + b_ref[0].reshape(1, 1, 1, cout)
    return jnp.maximum(acc, 0.0)


def _pool2(h):
    n, hh, ww, c = h.shape
    r = h.reshape(n, hh // 2, 2, ww // 2, 2, c)
    return jnp.max(jnp.max(r, axis=4), axis=2)


def _expert_body(perm_ref, x0, x1, x2, x3, x4, x5, x6, x7,
                 wb1_ref, b1_ref, w2_ref, b2_ref, w3_ref, b3_ref,
                 w4_ref, b4_ref, pw_ref, pb_ref, cw_ref, cb_ref, out_ref,
                 pad2, pad3, pad4):
    del perm_ref
    n = CHUNK
    zrow = jnp.zeros((1, 1, HW * IN_CH), jnp.float32)
    rows = []
    for xr in (x0, x1, x2, x3, x4, x5, x6, x7):
        xb = xr[0]
        row = jnp.concatenate([xb[0], xb[1], xb[2]], axis=1)
        rows.append(jnp.concatenate(
            [zrow, row.reshape(1, HW, HW * IN_CH), zrow], axis=1))
    xg = jnp.concatenate(rows, axis=0)
    acc = None
    for dy in range(3):
        sl = xg[:, dy:dy + HW, :].reshape(n * HW, HW * IN_CH)
        r = jnp.dot(sl, wb1_ref[0, dy], preferred_element_type=jnp.float32)
        acc = r if acc is None else acc + r
    h = jnp.maximum(acc + b1_ref[0], 0.0)
    h = h.reshape(n, HW, HW, 64)

    h = _conv_nhwc(pad2, h, w2_ref, b2_ref, 32, 64)
    h = _pool2(h)
    h = _conv_nhwc(pad3, h, w3_ref, b3_ref, 16, 128)
    h = _pool2(h)
    h = _conv_nhwc(pad4, h, w4_ref, b4_ref, 8, 256)
    z = jnp.mean(h.reshape(n, 64, 256), axis=1)
    z = jnp.dot(z, pw_ref[0], preferred_element_type=jnp.float32) + pb_ref[0]
    out_ref[...] = (
        jnp.dot(z, cw_ref[0], preferred_element_type=jnp.float32) + cb_ref[0]
    )


def _prep_conv(w, b, g, bb):
    scale = g / jnp.sqrt(1.0 + 1e-5)
    w = w * scale[:, :, None, None, None]
    b = b * scale + bb
    e, cout, cin, _, _ = w.shape
    wt = jnp.transpose(w, (0, 3, 4, 2, 1)).reshape(e, 9, cin, cout)
    return wt, b.reshape(e, 1, cout)


def kernel(x, sc_conv_w, sc_conv_b, sc_lin_w, sc_lin_b,
           enc1_w, enc1_b, bn1_g, bn1_b, enc2_w, enc2_b, bn2_g, bn2_b,
           enc3_w, enc3_b, bn3_g, bn3_b, enc4_w, enc4_b, bn4_g, bn4_b,
           proj_w, proj_b, cls_w, cls_b):
    f32 = jnp.float32

    w_sc = sc_conv_w.reshape(E * 8, IN_CH)
    b_sc = sc_conv_b.reshape(1, E * 8)
    lin = sc_lin_w.reshape(E, 8)
    lmat = (jnp.eye(E, dtype=f32)[:, None, :] * lin[:, :, None]).reshape(E * 8, E)
    lb = sc_lin_b.reshape(1, E)

    scores = pl.pallas_call(
        _scores_body,
        grid=(B // 32,),
        in_specs=[
            pl.BlockSpec((32, IN_CH, HW, HW), lambda i: (i, 0, 0, 0)),
            pl.BlockSpec((E * 8, IN_CH), lambda i: (0, 0)),
            pl.BlockSpec((1, E * 8), lambda i: (0, 0)),
            pl.BlockSpec((E * 8, E), lambda i: (0, 0)),
            pl.BlockSpec((1, E), lambda i: (0, 0)),
        ],
        out_specs=pl.BlockSpec((32, E), lambda i: (i, 0)),
        out_shape=jax.ShapeDtypeStruct((B, E), f32),
    )(x, w_sc, b_sc, lmat, lb)

    gk = jax.random.fold_in(jax.random.key(0), 123)
    u = jax.random.uniform(gk, (B, E), dtype=f32, minval=1e-6, maxval=1.0 - 1e-6)
    gumbel = -jnp.log(-jnp.log(u))

    slot, perm = pl.pallas_call(
        _route_body,
        out_shape=(jax.ShapeDtypeStruct((B, 1), jnp.int32),
                   jax.ShapeDtypeStruct((1, B), jnp.int32)),
    )(scores, gumbel)
    perm_flat = perm.reshape(B)

    w1p, b1p = _prep_conv(enc1_w, enc1_b, bn1_g, bn1_b)
    w1r = w1p.reshape(E, 3, 3, IN_CH, 64)
    xp_i = jnp.arange(HW)[None, :, None]
    x_i = jnp.arange(HW)[None, None, :]
    dx_i = jnp.arange(3)[:, None, None]
    sel = (xp_i == x_i + dx_i - 1).astype(f32)
    wb1 = jnp.einsum('dpx,eydco->eycpxo', sel, w1r).reshape(
        E, 3, HW * IN_CH, HW * 64)
    b1t = jnp.tile(b1p.reshape(E, 64), (1, HW)).reshape(E, 1, HW * 64)
    w2p, b2p = _prep_conv(enc2_w, enc2_b, bn2_g, bn2_b)
    w3p, b3p = _prep_conv(enc3_w, enc3_b, bn3_g, bn3_b)
    w4p, b4p = _prep_conv(enc4_w, enc4_b, bn4_g, bn4_b)
    pwt = jnp.transpose(proj_w, (0, 2, 1))
    pbp = proj_b.reshape(E, 1, 256)
    cwt = jnp.transpose(cls_w, (0, 2, 1))
    cbp = cls_b.reshape(E, 1, NUM_CLASSES)

    sp = STEPS_PER_EXPERT

    def _xspec(j):
        return pl.BlockSpec((1, IN_CH, HW, HW),
                            lambda c, p, j=j: (p[c * CHUNK + j], 0, 0, 0))

    def _wspec(shape3):
        return pl.BlockSpec((1,) + shape3, lambda c, p: (c // sp, 0, 0, 0))

    def _bspec(cout):
        return pl.BlockSpec((1, 1, cout), lambda c, p: (c // sp, 0, 0))

    logits_g = pl.pallas_call(
        _expert_body,
        grid_spec=pltpu.PrefetchScalarGridSpec(
            num_scalar_prefetch=1,
            grid=(N_CHUNKS,),
            in_specs=[
                _xspec(0), _xspec(1), _xspec(2), _xspec(3),
                _xspec(4), _xspec(5), _xspec(6), _xspec(7),
                _wspec((3, HW * IN_CH, HW * 64)), _bspec(HW * 64),
                _wspec((9, 64, 64)), _bspec(64),
                _wspec((9, 64, 128)), _bspec(128),
                _wspec((9, 128, 256)), _bspec(256),
                pl.BlockSpec((1, 256, 256), lambda c, p: (c // sp, 0, 0)),
                _bspec(256),
                pl.BlockSpec((1, 256, NUM_CLASSES), lambda c, p: (c // sp, 0, 0)),
                _bspec(NUM_CLASSES),
            ],
            out_specs=pl.BlockSpec((CHUNK, NUM_CLASSES), lambda c, p: (c, 0)),
            scratch_shapes=[
                pltpu.VMEM((CHUNK, HW + 2, HW + 2, 64), f32),
                pltpu.VMEM((CHUNK, 18, 18, 64), f32),
                pltpu.VMEM((CHUNK, 10, 10, 128), f32),
            ],
        ),
        out_shape=jax.ShapeDtypeStruct((B, NUM_CLASSES), f32),
    )(perm_flat, x, x, x, x, x, x, x, x,
      wb1, b1t, w2p, b2p, w3p, b3p, w4p, b4p, pwt, pbp, cwt, cbp)

    out = pl.pallas_call(
        _scatter_body,
        out_shape=jax.ShapeDtypeStruct((B, NUM_CLASSES), f32),
    )(slot, logits_g)
    return out


# speedup vs baseline: 2.5724x; 1.0930x over previous
import functools

import jax
import jax.numpy as jnp
from jax.experimental import pallas as pl
from jax.experimental.pallas import tpu as pltpu

E = 8
NUM_CLASSES = 10
IN_CH = 3
B = 256
HW = 32
T = 0.1
CAP = 32
CHUNK = 8
N_CHUNKS = B // CHUNK
STEPS_PER_EXPERT = CAP // CHUNK
ROUTE_ROUNDS = 32


def _scores_body(x_ref, w_ref, b_ref, lmat_ref, lb_ref, out_ref):
    xb = x_ref[...]
    n = xb.shape[0]
    xb = xb.reshape(n, IN_CH, HW * HW)
    h = jax.lax.dot_general(xb, w_ref[...], (((1,), (1,)), ((), ())),
                            preferred_element_type=jnp.float32)
    h = jnp.maximum(h + b_ref[...].reshape(1, 1, E * 8), 0.0)
    m = jnp.mean(h, axis=1)
    out_ref[...] = (
        jnp.dot(m, lmat_ref[...], preferred_element_type=jnp.float32)
        + lb_ref[...]
    )


def _route_body(s_ref, g_ref, slot_ref, perm_ref):
    st0 = (s_ref[...] + g_ref[...]) / T
    ltri = (jax.lax.broadcasted_iota(jnp.int32, (B, B), 0)
            >= jax.lax.broadcasted_iota(jnp.int32, (B, B), 1)).astype(jnp.float32)
    e_iota = jax.lax.broadcasted_iota(jnp.int32, (1, E), 1)

    def body(it, state):
        assignment, cap, active = state
        deficit = jnp.clip(cap.astype(jnp.float32) / CAP, 0.0, 1.0)
        stt = st0 * (1.0 - deficit)
        full = cap >= CAP
        stt = jnp.where(full, -jnp.inf, stt)
        mx = jnp.max(stt, axis=1, keepdims=True)
        ismax = stt == mx
        best = jnp.min(jnp.where(ismax, e_iota, E), axis=1, keepdims=True)
        onehot = (e_iota == best).astype(jnp.float32) * active
        order = jnp.dot(ltri, onehot, preferred_element_type=jnp.float32)
        quota = jnp.minimum(jnp.int32(1) << jnp.minimum(it, 30), jnp.int32(CAP))
        space = jnp.minimum(jnp.int32(CAP) - cap, quota).astype(jnp.float32)
        sel = onehot * (order <= space).astype(jnp.float32)
        assignment = jnp.maximum(assignment, sel)
        cap = cap + jnp.sum(sel, axis=0, keepdims=True).astype(jnp.int32)
        active = active * (1.0 - jnp.max(sel, axis=1, keepdims=True))
        return assignment, cap, active

    init = (jnp.zeros((B, E), jnp.float32),
            jnp.zeros((1, E), jnp.int32),
            jnp.ones((B, 1), jnp.float32))
    assignment, _, _ = jax.lax.fori_loop(0, ROUTE_ROUNDS, body, init)

    order = jnp.dot(ltri, assignment, preferred_element_type=jnp.float32)
    eid = jnp.sum(assignment * e_iota.astype(jnp.float32), axis=1, keepdims=True)
    rank = jnp.sum(assignment * order, axis=1, keepdims=True) - 1.0
    slot = (eid * CAP + rank).astype(jnp.int32)
    slot_ref[...] = slot
    row_i = jax.lax.broadcasted_iota(jnp.int32, (B, B), 0).astype(jnp.float32)
    col_p = jax.lax.broadcasted_iota(jnp.int32, (B, B), 1)
    onehot_slot = (slot == col_p).astype(jnp.float32)
    perm_ref[...] = jnp.sum(onehot_slot * row_i, axis=0, keepdims=True).astype(jnp.int32)


def _scatter_body(slot_ref, lg_ref, out_ref):
    cols = jax.lax.broadcasted_iota(jnp.int32, (B, B), 1)
    onehot = (slot_ref[...] == cols).astype(jnp.float32)
    out_ref[...] = jnp.dot(onehot, lg_ref[...], preferred_element_type=jnp.float32)


def _conv_nhwc(pad_ref, h, w_ref, b_ref, hh, cout):
    n, _, _, cin = h.shape

    @pl.when(pl.program_id(0) == 0)
    def _zero_borders():
        pad_ref[:, 0:1, :, :] = jnp.zeros((n, 1, hh + 2, cin), jnp.float32)
        pad_ref[:, hh + 1:hh + 2, :, :] = jnp.zeros((n, 1, hh + 2, cin), jnp.float32)
        pad_ref[:, 1:hh + 1, 0:1, :] = jnp.zeros((n, hh, 1, cin), jnp.float32)
        pad_ref[:, 1:hh + 1, hh + 1:hh + 2, :] = jnp.zeros((n, hh, 1, cin), jnp.float32)

    pad_ref[:, 1:hh + 1, 1:hh + 1, :] = h
    w = w_ref[0]
    acc = None
    for dx in range(3):
        sx = pad_ref[:, :, dx:dx + hh, :]
        for dy in range(3):
            sl = sx[:, dy:dy + hh, :, :].reshape(n * hh * hh, cin)
            r = jnp.dot(sl, w[3 * dy + dx], preferred_element_type=jnp.float32)
            acc = r if acc is None else acc + r
    acc = acc.reshape(n, hh, hh, cout) + b_ref[0].reshape(1, 1, 1, cout)
    return jnp.maximum(acc, 0.0)


def _sel_mat(m, k):
    r = jax.lax.broadcasted_iota(jnp.int32, (k, m), 0)
    cidx = jax.lax.broadcasted_iota(jnp.int32, (k, m), 1)
    return (cidx == 2 * r).astype(jnp.float32)


def _pool2(h):
    n, hh, ww, c = h.shape
    aw = jnp.maximum(h[:, :, 0:ww - 1, :], h[:, :, 1:ww, :])
    sw = jnp.broadcast_to(_sel_mat(ww - 1, ww // 2)[None], (n, ww // 2, ww - 1))
    t = jax.lax.dot_general(sw, aw, (((2,), (2,)), ((0,), (0,))),
                            preferred_element_type=jnp.float32)
    ah = jnp.maximum(t[:, :, 0:hh - 1, :], t[:, :, 1:hh, :])
    sh = jnp.broadcast_to(_sel_mat(hh - 1, hh // 2)[None], (n, hh // 2, hh - 1))
    return jax.lax.dot_general(sh, ah, (((2,), (2,)), ((0,), (0,))),
                               preferred_element_type=jnp.float32)


def _expert_body(perm_ref, x0, x1, x2, x3, x4, x5, x6, x7,
                 wb1_ref, b1_ref, w2_ref, b2_ref, w3_ref, b3_ref,
                 w4_ref, b4_ref, pw_ref, pb_ref, cw_ref, cb_ref, out_ref,
                 pad2, pad3, pad4):
    del perm_ref
    n = CHUNK
    zrow = jnp.zeros((1, 1, HW * IN_CH), jnp.float32)
    rows = []
    for xr in (x0, x1, x2, x3, x4, x5, x6, x7):
        xb = xr[0]
        row = jnp.concatenate([xb[0], xb[1], xb[2]], axis=1)
        rows.append(jnp.concatenate(
            [zrow, row.reshape(1, HW, HW * IN_CH), zrow], axis=1))
    xg = jnp.concatenate(rows, axis=0)
    acc = None
    for dy in range(3):
        sl = xg[:, dy:dy + HW, :].reshape(n * HW, HW * IN_CH)
        r = jnp.dot(sl, wb1_ref[0, dy], preferred_element_type=jnp.float32)
        acc = r if acc is None else acc + r
    h = jnp.maximum(acc + b1_ref[0], 0.0)
    h = h.reshape(n, HW, HW, 64)

    h = _conv_nhwc(pad2, h, w2_ref, b2_ref, 32, 64)
    h = _pool2(h)
    h = _conv_nhwc(pad3, h, w3_ref, b3_ref, 16, 128)
    h = _pool2(h)
    h = _conv_nhwc(pad4, h, w4_ref, b4_ref, 8, 256)
    z = jnp.mean(h.reshape(n, 64, 256), axis=1)
    z = jnp.dot(z, pw_ref[0], preferred_element_type=jnp.float32) + pb_ref[0]
    out_ref[...] = (
        jnp.dot(z, cw_ref[0], preferred_element_type=jnp.float32) + cb_ref[0]
    )


def _prep_conv(w, b, g, bb):
    scale = g / jnp.sqrt(1.0 + 1e-5)
    w = w * scale[:, :, None, None, None]
    b = b * scale + bb
    e, cout, cin, _, _ = w.shape
    wt = jnp.transpose(w, (0, 3, 4, 2, 1)).reshape(e, 9, cin, cout)
    return wt, b.reshape(e, 1, cout)


def kernel(x, sc_conv_w, sc_conv_b, sc_lin_w, sc_lin_b,
           enc1_w, enc1_b, bn1_g, bn1_b, enc2_w, enc2_b, bn2_g, bn2_b,
           enc3_w, enc3_b, bn3_g, bn3_b, enc4_w, enc4_b, bn4_g, bn4_b,
           proj_w, proj_b, cls_w, cls_b):
    f32 = jnp.float32

    w_sc = sc_conv_w.reshape(E * 8, IN_CH)
    b_sc = sc_conv_b.reshape(1, E * 8)
    lin = sc_lin_w.reshape(E, 8)
    lmat = (jnp.eye(E, dtype=f32)[:, None, :] * lin[:, :, None]).reshape(E * 8, E)
    lb = sc_lin_b.reshape(1, E)

    scores = pl.pallas_call(
        _scores_body,
        grid=(B // 32,),
        in_specs=[
            pl.BlockSpec((32, IN_CH, HW, HW), lambda i: (i, 0, 0, 0)),
            pl.BlockSpec((E * 8, IN_CH), lambda i: (0, 0)),
            pl.BlockSpec((1, E * 8), lambda i: (0, 0)),
            pl.BlockSpec((E * 8, E), lambda i: (0, 0)),
            pl.BlockSpec((1, E), lambda i: (0, 0)),
        ],
        out_specs=pl.BlockSpec((32, E), lambda i: (i, 0)),
        out_shape=jax.ShapeDtypeStruct((B, E), f32),
    )(x, w_sc, b_sc, lmat, lb)

    gk = jax.random.fold_in(jax.random.key(0), 123)
    u = jax.random.uniform(gk, (B, E), dtype=f32, minval=1e-6, maxval=1.0 - 1e-6)
    gumbel = -jnp.log(-jnp.log(u))

    slot, perm = pl.pallas_call(
        _route_body,
        out_shape=(jax.ShapeDtypeStruct((B, 1), jnp.int32),
                   jax.ShapeDtypeStruct((1, B), jnp.int32)),
    )(scores, gumbel)
    perm_flat = perm.reshape(B)

    w1p, b1p = _prep_conv(enc1_w, enc1_b, bn1_g, bn1_b)
    w1r = w1p.reshape(E, 3, 3, IN_CH, 64)
    xp_i = jnp.arange(HW)[None, :, None]
    x_i = jnp.arange(HW)[None, None, :]
    dx_i = jnp.arange(3)[:, None, None]
    sel = (xp_i == x_i + dx_i - 1).astype(f32)
    wb1 = jnp.einsum('dpx,eydco->eycpxo', sel, w1r).reshape(
        E, 3, HW * IN_CH, HW * 64)
    b1t = jnp.tile(b1p.reshape(E, 64), (1, HW)).reshape(E, 1, HW * 64)
    w2p, b2p = _prep_conv(enc2_w, enc2_b, bn2_g, bn2_b)
    w3p, b3p = _prep_conv(enc3_w, enc3_b, bn3_g, bn3_b)
    w4p, b4p = _prep_conv(enc4_w, enc4_b, bn4_g, bn4_b)
    pwt = jnp.transpose(proj_w, (0, 2, 1))
    pbp = proj_b.reshape(E, 1, 256)
    cwt = jnp.transpose(cls_w, (0, 2, 1))
    cbp = cls_b.reshape(E, 1, NUM_CLASSES)

    sp = STEPS_PER_EXPERT

    def _xspec(j):
        return pl.BlockSpec((1, IN_CH, HW, HW),
                            lambda c, p, j=j: (p[c * CHUNK + j], 0, 0, 0))

    def _wspec(shape3):
        return pl.BlockSpec((1,) + shape3, lambda c, p: (c // sp, 0, 0, 0))

    def _bspec(cout):
        return pl.BlockSpec((1, 1, cout), lambda c, p: (c // sp, 0, 0))

    logits_g = pl.pallas_call(
        _expert_body,
        grid_spec=pltpu.PrefetchScalarGridSpec(
            num_scalar_prefetch=1,
            grid=(N_CHUNKS,),
            in_specs=[
                _xspec(0), _xspec(1), _xspec(2), _xspec(3),
                _xspec(4), _xspec(5), _xspec(6), _xspec(7),
                _wspec((3, HW * IN_CH, HW * 64)), _bspec(HW * 64),
                _wspec((9, 64, 64)), _bspec(64),
                _wspec((9, 64, 128)), _bspec(128),
                _wspec((9, 128, 256)), _bspec(256),
                pl.BlockSpec((1, 256, 256), lambda c, p: (c // sp, 0, 0)),
                _bspec(256),
                pl.BlockSpec((1, 256, NUM_CLASSES), lambda c, p: (c // sp, 0, 0)),
                _bspec(NUM_CLASSES),
            ],
            out_specs=pl.BlockSpec((CHUNK, NUM_CLASSES), lambda c, p: (c, 0)),
            scratch_shapes=[
                pltpu.VMEM((CHUNK, HW + 2, HW + 2, 64), f32),
                pltpu.VMEM((CHUNK, 18, 18, 64), f32),
                pltpu.VMEM((CHUNK, 10, 10, 128), f32),
            ],
        ),
        out_shape=jax.ShapeDtypeStruct((B, NUM_CLASSES), f32),
    )(perm_flat, x, x, x, x, x, x, x, x,
      wb1, b1t, w2p, b2p, w3p, b3p, w4p, b4p, pwt, pbp, cwt, cbp)

    out = pl.pallas_call(
        _scatter_body,
        out_shape=jax.ShapeDtypeStruct((B, NUM_CLASSES), f32),
    )(slot, logits_g)
    return out
